# baseline (speedup 1.0000x reference)
import functools
import math

import jax
import jax.numpy as jnp
from jax import lax
from jax.experimental import pallas as pl
from jax.experimental.pallas import tpu as pltpu
from jax.experimental.pallas import tpu_sc as plsc

N = 10000
E = 320000
D = 128
H = 8
C = 16
NPAD = 10048
ROWW = 136
NW = 32
B = 32
NB = 328
E_PAD = NW * NB * B
DUMMY = NPAD - 8
BLK = 1256


def _proj_body(x_ref, wl_ref, bl_ref, wr_ref, br_ref, xl_ref, xr_ref):
    xv = x_ref[...]
    xl_ref[...] = (
        jnp.dot(xv, wl_ref[...], preferred_element_type=jnp.float32) + bl_ref[...]
    )
    xr_ref[...] = (
        jnp.dot(xv, wr_ref[...], preferred_element_type=jnp.float32) + br_ref[...]
    )


def _proj(xpad, Wl, bl2, Wr, br2):
    return pl.pallas_call(
        _proj_body,
        grid=(NPAD // BLK,),
        in_specs=[
            pl.BlockSpec((BLK, D), lambda i: (i, 0)),
            pl.BlockSpec((D, D), lambda i: (0, 0)),
            pl.BlockSpec((1, D), lambda i: (0, 0)),
            pl.BlockSpec((D, D), lambda i: (0, 0)),
            pl.BlockSpec((1, D), lambda i: (0, 0)),
        ],
        out_specs=[
            pl.BlockSpec((BLK, D), lambda i: (i, 0)),
            pl.BlockSpec((BLK, D), lambda i: (i, 0)),
        ],
        out_shape=[
            jax.ShapeDtypeStruct((NPAD, D), jnp.float32),
            jax.ShapeDtypeStruct((NPAD, D), jnp.float32),
        ],
    )(xpad, Wl, bl2, Wr, br2)


def _edge_body(xl_h, xr_h, att_h, se_h, zeros_h, out_h,
               acc_sh, idx_i, xlb0, xlb1, xlb2, xlb3,
               xrb0, xrb1, xrb2, xrb3, msg, attv,
               sem_i, sem_g, sem_s):
    c = lax.axis_index("c")
    s = lax.axis_index("s")
    wid = s * 2 + c
    tr = NPAD // 16
    base_e = wid * (NB * B)
    lanes = lax.iota(jnp.int32, 16)
    zero16 = jnp.zeros((16,), jnp.float32)
    rows = ((xlb0, xrb0), (xlb1, xrb1), (xlb2, xrb2), (xlb3, xrb3))

    def _idx_start(b, slot):
        off = base_e + b * B
        pltpu.async_copy(
            se_h.at[:, pl.ds(off, B)], idx_i.at[slot], sem_i.at[slot])

    def _idx_wait(b, slot):
        off = base_e + b * B
        pltpu.make_async_copy(
            se_h.at[:, pl.ds(off, B)], idx_i.at[slot], sem_i.at[slot]).wait()

    def _gather_start(slot, p):
        rxl, rxr = rows[p]
        pltpu.async_copy(xl_h.at[idx_i.at[slot, 0]], rxl, sem_g.at[p, 0])
        pltpu.async_copy(xr_h.at[idx_i.at[slot, 1]], rxr, sem_g.at[p, 1])

    def _gather_wait(slot, p):
        rxl, rxr = rows[p]
        pltpu.make_async_copy(
            xl_h.at[idx_i.at[slot, 0]], rxl, sem_g.at[p, 0]).wait()
        pltpu.make_async_copy(
            xr_h.at[idx_i.at[slot, 1]], rxr, sem_g.at[p, 1]).wait()

    def _scatter_start(slot):
        pltpu.async_copy(msg, acc_sh.at[idx_i.at[slot, 1]], sem_s, add=True)

    def _scatter_wait(slot):
        pltpu.make_async_copy(msg, acc_sh.at[idx_i.at[slot, 1]], sem_s).wait()

    def _compute(p):
        rxl, rxr = rows[p]

        def _head(h, hcarry):
            avecs = [attv[h * C + cc] for cc in range(C)]
            wcol = jnp.full((16,), 128 + h, jnp.int32)
            hc = h * C

            def _grp(g, gcarry):
                eidx = g * 16 + lanes
                accs = [zero16, zero16, zero16, zero16]
                xs = []
                cols = []
                for cc in range(C):
                    col = jnp.bitwise_and(lanes + cc, 15) + hc
                    a = plsc.load_gather(rxl, [eidx, col])
                    bb = plsc.load_gather(rxr, [eidx, col])
                    u = a + bb
                    u = jnp.maximum(u, 0.2 * u)
                    accs[cc % 4] = accs[cc % 4] + u * avecs[cc]
                    xs.append(a)
                    cols.append(col)
                w = jnp.exp((accs[0] + accs[1]) + (accs[2] + accs[3]))
                plsc.store_scatter(msg, [eidx, wcol], w)
                for cc in range(C):
                    plsc.store_scatter(msg, [eidx, cols[cc]], xs[cc] * w)
                return gcarry

            return lax.fori_loop(0, B // 16, _grp, hcarry)

        lax.fori_loop(0, H, _head, 0)

    for k in range(4):
        _idx_start(k, k)
    pltpu.sync_copy(att_h, attv)
    pltpu.sync_copy(zeros_h, acc_sh.at[pl.ds(s * tr, tr)])
    for k in range(3):
        _idx_wait(k, k)
        _gather_start(k, k)
    plsc.subcore_barrier()

    def _iter(i, carry):
        for par in range(8):
            b = i * 8 + par
            p = par % 4

            @pl.when(b + 3 < NB)
            def _():
                _idx_wait(b + 3, (par + 3) % 8)
                _gather_start((par + 3) % 8, (par + 3) % 4)

            _gather_wait(par, p)

            @pl.when(b > 0)
            def _():
                _scatter_wait((par + 7) % 8)

            _compute(p)
            _scatter_start(par)

            @pl.when(b + 4 < NB)
            def _():
                _idx_start(b + 4, (par + 4) % 8)

        return carry

    lax.fori_loop(0, NB // 8, _iter, 0)

    _scatter_wait((NB - 1) % 8)
    plsc.subcore_barrier()
    pltpu.sync_copy(
        acc_sh.at[pl.ds(s * tr, tr)],
        out_h.at[c, pl.ds(s * tr, tr)],
    )


def _edge_pass(xl, xr, attf, se, zeros):
    mesh = plsc.VectorSubcoreMesh(core_axis_name="c", subcore_axis_name="s")
    kern = pl.kernel(
        _edge_body,
        out_type=jax.ShapeDtypeStruct((2, NPAD, ROWW), jnp.float32),
        mesh=mesh,
        scratch_types=[
            pltpu.VMEM_SHARED((NPAD, ROWW), jnp.float32),
            pltpu.VMEM((8, 2, B), jnp.int32),
            pltpu.VMEM((B, D), jnp.float32),
            pltpu.VMEM((B, D), jnp.float32),
            pltpu.VMEM((B, D), jnp.float32),
            pltpu.VMEM((B, D), jnp.float32),
            pltpu.VMEM((B, D), jnp.float32),
            pltpu.VMEM((B, D), jnp.float32),
            pltpu.VMEM((B, D), jnp.float32),
            pltpu.VMEM((B, D), jnp.float32),
            pltpu.VMEM((B, ROWW), jnp.float32),
            pltpu.VMEM((D, 16), jnp.float32),
            pltpu.SemaphoreType.DMA((8,)),
            pltpu.SemaphoreType.DMA((4, 2)),
            pltpu.SemaphoreType.DMA,
        ],
        compiler_params=pltpu.CompilerParams(
            needs_layout_passes=False, use_tc_tiling_on_sc=False),
    )
    return kern(xl, xr, attf, se, zeros)


def _epi_body(agg_ref, den_ref, xi_ref, wres_ref, bres_ref, gb_ref,
              exp_ref, gam_ref, bln_ref, out_ref):
    a = agg_ref[0] + agg_ref[1]
    d8 = den_ref[0] + den_ref[1]
    dfull = jnp.dot(d8, exp_ref[...], preferred_element_type=jnp.float32)
    gat = a / (dfull + 1e-16) + gb_ref[...]
    res = (
        jnp.dot(xi_ref[...], wres_ref[...], preferred_element_type=jnp.float32)
        + bres_ref[...]
    )
    y = gat + res
    mu = jnp.mean(y, axis=-1, keepdims=True)
    yc = y - mu
    var = jnp.mean(yc * yc, axis=-1, keepdims=True)
    yn = yc * lax.rsqrt(var + 1e-5)
    yn = yn * gam_ref[...] + bln_ref[...]
    out_ref[...] = 0.5 * yn * (1.0 + lax.erf(yn * (1.0 / math.sqrt(2.0))))


def _epilogue(agg, den, xipad, wres_eff, bres_eff, gb2, expand, gam2, bln2):
    return pl.pallas_call(
        _epi_body,
        grid=(NPAD // BLK,),
        in_specs=[
            pl.BlockSpec((2, BLK, D), lambda i: (0, i, 0)),
            pl.BlockSpec((2, BLK, H), lambda i: (0, i, 0)),
            pl.BlockSpec((BLK, D), lambda i: (i, 0)),
            pl.BlockSpec((D, D), lambda i: (0, 0)),
            pl.BlockSpec((1, D), lambda i: (0, 0)),
            pl.BlockSpec((1, D), lambda i: (0, 0)),
            pl.BlockSpec((H, D), lambda i: (0, 0)),
            pl.BlockSpec((1, D), lambda i: (0, 0)),
            pl.BlockSpec((1, D), lambda i: (0, 0)),
        ],
        out_specs=pl.BlockSpec((BLK, D), lambda i: (i, 0)),
        out_shape=jax.ShapeDtypeStruct((NPAD, D), jnp.float32),
    )(agg, den, xipad, wres_eff, bres_eff, gb2, expand, gam2, bln2)


def kernel(x, x_initial, edge_index, Wl, bl, Wr, br, att, gat_bias,
           Wres, bres, beta, gamma, beta_ln):
    f32 = jnp.float32
    xpad = jnp.zeros((NPAD, D), f32).at[:N].set(x)
    xipad = jnp.zeros((NPAD, D), f32).at[:N].set(x_initial)

    loop = jnp.arange(N, dtype=jnp.int32)
    npad_e = E_PAD - (E + N)
    src = jnp.concatenate(
        [edge_index[0], loop, jnp.full((npad_e,), DUMMY, jnp.int32)])
    dst = jnp.concatenate(
        [edge_index[1], loop, jnp.full((npad_e,), DUMMY, jnp.int32)])
    se = jnp.stack([src, dst])

    xl, xr = _proj(xpad, Wl, bl.reshape(1, D), Wr, br.reshape(1, D))

    rot = (jnp.arange(C)[:, None] + jnp.arange(16)[None, :]) % C
    attf = att[:, rot].reshape(D, 16)
    zeros = jnp.zeros((NPAD // 16, ROWW), f32)
    acc = _edge_pass(xl, xr, attf, se, zeros)

    agg = acc[:, :, :D]
    den = acc[:, :, D:D + H]

    expand = jnp.repeat(jnp.eye(H, dtype=f32), C, axis=1)
    wres_eff = Wres * beta
    bres_eff = (bres * beta).reshape(1, D)

    y = _epilogue(agg, den, xipad, wres_eff, bres_eff,
                  gat_bias.reshape(1, D), expand,
                  gamma.reshape(1, D), beta_ln.reshape(1, D))
    return y[:N]


# speedup vs baseline: 1.1671x; 1.1671x over previous
import functools
import math

import jax
import jax.numpy as jnp
from jax import lax
from jax.experimental import pallas as pl
from jax.experimental.pallas import tpu as pltpu
from jax.experimental.pallas import tpu_sc as plsc

N = 10000
E = 320000
D = 128
H = 8
C = 16
NPAD = 10048
ROWW = 136
NW = 32
B = 64
NB = 164
E_PAD = NW * NB * B
DUMMY = NPAD - 8
BLK = 1256


def _proj_body(x_ref, wl_ref, bl_ref, wr_ref, br_ref, xl_ref, xr_ref):
    xv = x_ref[...]
    xl_ref[...] = (
        jnp.dot(xv, wl_ref[...], preferred_element_type=jnp.float32) + bl_ref[...]
    ).astype(jnp.bfloat16)
    xr_ref[...] = (
        jnp.dot(xv, wr_ref[...], preferred_element_type=jnp.float32) + br_ref[...]
    ).astype(jnp.bfloat16)


def _proj(xpad, Wl, bl2, Wr, br2):
    return pl.pallas_call(
        _proj_body,
        grid=(NPAD // BLK,),
        in_specs=[
            pl.BlockSpec((BLK, D), lambda i: (i, 0)),
            pl.BlockSpec((D, D), lambda i: (0, 0)),
            pl.BlockSpec((1, D), lambda i: (0, 0)),
            pl.BlockSpec((D, D), lambda i: (0, 0)),
            pl.BlockSpec((1, D), lambda i: (0, 0)),
        ],
        out_specs=[
            pl.BlockSpec((BLK, D), lambda i: (i, 0)),
            pl.BlockSpec((BLK, D), lambda i: (i, 0)),
        ],
        out_shape=[
            jax.ShapeDtypeStruct((NPAD, D), jnp.bfloat16),
            jax.ShapeDtypeStruct((NPAD, D), jnp.bfloat16),
        ],
    )(xpad, Wl, bl2, Wr, br2)


def _edge_body(xl_h, xr_h, att_h, se_h, zeros_h, out_h,
               acc_sh, idx_i, xlb0, xlb1, xrb0, xrb1, msg, attv,
               sem_i, sem_g, sem_s):
    c = lax.axis_index("c")
    s = lax.axis_index("s")
    wid = s * 2 + c
    tr = NPAD // 16
    base_e = wid * (NB * B)
    lanes = lax.iota(jnp.int32, 16)
    zero16 = jnp.zeros((16,), jnp.float32)
    rows = ((xlb0, xrb0), (xlb1, xrb1))

    def _idx_start(b, slot):
        off = base_e + b * B
        pltpu.async_copy(
            se_h.at[:, pl.ds(off, B)], idx_i.at[slot], sem_i.at[slot])

    def _idx_wait(b, slot):
        off = base_e + b * B
        pltpu.make_async_copy(
            se_h.at[:, pl.ds(off, B)], idx_i.at[slot], sem_i.at[slot]).wait()

    HB = B // 2

    def _gather_start(slot, p):
        rxl, rxr = rows[p]
        for k in range(2):
            pltpu.async_copy(
                xl_h.at[idx_i.at[slot, 0, pl.ds(k * HB, HB)]],
                rxl.at[pl.ds(k * HB, HB)], sem_g.at[p, k])
            pltpu.async_copy(
                xr_h.at[idx_i.at[slot, 1, pl.ds(k * HB, HB)]],
                rxr.at[pl.ds(k * HB, HB)], sem_g.at[p, 2 + k])

    def _gather_wait(slot, p):
        rxl, rxr = rows[p]
        for k in range(2):
            pltpu.make_async_copy(
                xl_h.at[idx_i.at[slot, 0, pl.ds(k * HB, HB)]],
                rxl.at[pl.ds(k * HB, HB)], sem_g.at[p, k]).wait()
            pltpu.make_async_copy(
                xr_h.at[idx_i.at[slot, 1, pl.ds(k * HB, HB)]],
                rxr.at[pl.ds(k * HB, HB)], sem_g.at[p, 2 + k]).wait()

    def _scatter_start(slot):
        pltpu.async_copy(msg, acc_sh.at[idx_i.at[slot, 1]], sem_s, add=True)

    def _scatter_wait(slot):
        pltpu.make_async_copy(msg, acc_sh.at[idx_i.at[slot, 1]], sem_s).wait()

    def _compute(p):
        rxl, rxr = rows[p]

        def _head(h, hcarry):
            ae = [attv[h * C + 2 * k] for k in range(8)]
            ao = [attv[h * C + 2 * k + 1] for k in range(8)]
            wcol = jnp.full((16,), 128 + h, jnp.int32)
            hc = h * C

            def _grp(g, gcarry):
                eidx = g * 16 + lanes
                accs = [zero16, zero16, zero16, zero16]
                xs = []
                cols = []
                for k in range(8):
                    c8 = jnp.bitwise_and(lanes + k, 7)
                    coli = c8 + 8 * h
                    vl = plsc.load_gather(rxl, [eidx, coli])
                    vr = plsc.load_gather(rxr, [eidx, coli])
                    le, lo = plsc.unpack(
                        plsc.bitcast(vl, jnp.bfloat16),
                        format=plsc.PackFormat.INTERLEAVED)
                    re_, ro = plsc.unpack(
                        plsc.bitcast(vr, jnp.bfloat16),
                        format=plsc.PackFormat.INTERLEAVED)
                    ue = le + re_
                    ue = jnp.maximum(ue, 0.2 * ue)
                    uo = lo + ro
                    uo = jnp.maximum(uo, 0.2 * uo)
                    accs[k % 4] = accs[k % 4] + ue * ae[k]
                    accs[(k + 2) % 4] = accs[(k + 2) % 4] + uo * ao[k]
                    ce = 2 * c8 + hc
                    xs.append((le, lo))
                    cols.append(ce)
                w = jnp.exp((accs[0] + accs[1]) + (accs[2] + accs[3]))
                plsc.store_scatter(msg, [eidx, wcol], w)
                for k in range(8):
                    le, lo = xs[k]
                    ce = cols[k]
                    plsc.store_scatter(msg, [eidx, ce], le * w)
                    plsc.store_scatter(msg, [eidx, ce + 1], lo * w)
                return gcarry

            return lax.fori_loop(0, B // 16, _grp, hcarry)

        lax.fori_loop(0, H, _head, 0)

    _idx_start(0, 0)
    _idx_start(1, 1)
    pltpu.sync_copy(att_h, attv)
    pltpu.sync_copy(zeros_h, acc_sh.at[pl.ds(s * tr, tr)])
    _idx_wait(0, 0)
    _gather_start(0, 0)
    plsc.subcore_barrier()

    def _iter(i, carry):
        for par in range(4):
            b = i * 4 + par
            p = par & 1

            @pl.when(b + 1 < NB)
            def _():
                _idx_wait(b + 1, (par + 1) % 4)
                _gather_start((par + 1) % 4, 1 - p)

            _gather_wait(par, p)

            @pl.when(b > 0)
            def _():
                _scatter_wait((par + 3) % 4)

            _compute(p)
            _scatter_start(par)

            @pl.when(b + 2 < NB)
            def _():
                _idx_start(b + 2, (par + 2) % 4)

        return carry

    lax.fori_loop(0, NB // 4, _iter, 0)

    _scatter_wait((NB - 1) % 4)
    plsc.subcore_barrier()
    pltpu.sync_copy(
        acc_sh.at[pl.ds(s * tr, tr)],
        out_h.at[c, pl.ds(s * tr, tr)],
    )


def _edge_pass(xl, xr, attf, se, zeros):
    mesh = plsc.VectorSubcoreMesh(core_axis_name="c", subcore_axis_name="s")
    kern = pl.kernel(
        _edge_body,
        out_type=jax.ShapeDtypeStruct((2, NPAD, ROWW), jnp.float32),
        mesh=mesh,
        scratch_types=[
            pltpu.VMEM_SHARED((NPAD, ROWW), jnp.float32),
            pltpu.VMEM((4, 2, B), jnp.int32),
            pltpu.VMEM((B, D // 2), jnp.int32),
            pltpu.VMEM((B, D // 2), jnp.int32),
            pltpu.VMEM((B, D // 2), jnp.int32),
            pltpu.VMEM((B, D // 2), jnp.int32),
            pltpu.VMEM((B, ROWW), jnp.float32),
            pltpu.VMEM((D, 16), jnp.float32),
            pltpu.SemaphoreType.DMA((4,)),
            pltpu.SemaphoreType.DMA((2, 4)),
            pltpu.SemaphoreType.DMA,
        ],
        compiler_params=pltpu.CompilerParams(
            needs_layout_passes=False, use_tc_tiling_on_sc=False),
    )
    return kern(xl, xr, attf, se, zeros)


def _epi_body(agg_ref, den_ref, xi_ref, wres_ref, bres_ref, gb_ref,
              exp_ref, gam_ref, bln_ref, out_ref):
    a = agg_ref[0] + agg_ref[1]
    d8 = den_ref[0] + den_ref[1]
    dfull = jnp.dot(d8, exp_ref[...], preferred_element_type=jnp.float32)
    gat = a / (dfull + 1e-16) + gb_ref[...]
    res = (
        jnp.dot(xi_ref[...], wres_ref[...], preferred_element_type=jnp.float32)
        + bres_ref[...]
    )
    y = gat + res
    mu = jnp.mean(y, axis=-1, keepdims=True)
    yc = y - mu
    var = jnp.mean(yc * yc, axis=-1, keepdims=True)
    yn = yc * lax.rsqrt(var + 1e-5)
    yn = yn * gam_ref[...] + bln_ref[...]
    out_ref[...] = 0.5 * yn * (1.0 + lax.erf(yn * (1.0 / math.sqrt(2.0))))


def _epilogue(agg, den, xipad, wres_eff, bres_eff, gb2, expand, gam2, bln2):
    return pl.pallas_call(
        _epi_body,
        grid=(NPAD // BLK,),
        in_specs=[
            pl.BlockSpec((2, BLK, D), lambda i: (0, i, 0)),
            pl.BlockSpec((2, BLK, H), lambda i: (0, i, 0)),
            pl.BlockSpec((BLK, D), lambda i: (i, 0)),
            pl.BlockSpec((D, D), lambda i: (0, 0)),
            pl.BlockSpec((1, D), lambda i: (0, 0)),
            pl.BlockSpec((1, D), lambda i: (0, 0)),
            pl.BlockSpec((H, D), lambda i: (0, 0)),
            pl.BlockSpec((1, D), lambda i: (0, 0)),
            pl.BlockSpec((1, D), lambda i: (0, 0)),
        ],
        out_specs=pl.BlockSpec((BLK, D), lambda i: (i, 0)),
        out_shape=jax.ShapeDtypeStruct((NPAD, D), jnp.float32),
    )(agg, den, xipad, wres_eff, bres_eff, gb2, expand, gam2, bln2)


def kernel(x, x_initial, edge_index, Wl, bl, Wr, br, att, gat_bias,
           Wres, bres, beta, gamma, beta_ln):
    f32 = jnp.float32
    xpad = jnp.zeros((NPAD, D), f32).at[:N].set(x)
    xipad = jnp.zeros((NPAD, D), f32).at[:N].set(x_initial)

    loop = jnp.arange(N, dtype=jnp.int32)
    npad_e = E_PAD - (E + N)
    src = jnp.concatenate(
        [edge_index[0], loop, jnp.full((npad_e,), DUMMY, jnp.int32)])
    dst = jnp.concatenate(
        [edge_index[1], loop, jnp.full((npad_e,), DUMMY, jnp.int32)])
    se = jnp.stack([src, dst])

    xl_bf, xr_bf = _proj(xpad, Wl, bl.reshape(1, D), Wr, br.reshape(1, D))
    xl = jax.lax.bitcast_convert_type(
        xl_bf.reshape(NPAD, D // 2, 2), jnp.int32)
    xr = jax.lax.bitcast_convert_type(
        xr_bf.reshape(NPAD, D // 2, 2), jnp.int32)

    rot8 = (jnp.arange(8)[:, None] + jnp.arange(16)[None, :]) % 8
    att_e = att[:, 2 * rot8]
    att_o = att[:, 2 * rot8 + 1]
    attf = jnp.stack([att_e, att_o], axis=2).reshape(D, 16)
    zeros = jnp.zeros((NPAD // 16, ROWW), f32)
    acc = _edge_pass(xl, xr, attf, se, zeros)

    agg = acc[:, :, :D]
    den = acc[:, :, D:D + H]

    expand = jnp.repeat(jnp.eye(H, dtype=f32), C, axis=1)
    wres_eff = Wres * beta
    bres_eff = (bres * beta).reshape(1, D)

    y = _epilogue(agg, den, xipad, wres_eff, bres_eff,
                  gat_bias.reshape(1, D), expand,
                  gamma.reshape(1, D), beta_ln.reshape(1, D))
    return y[:N]


# speedup vs baseline: 1.2132x; 1.0395x over previous
import functools
import math

import jax
import jax.numpy as jnp
from jax import lax
from jax.experimental import pallas as pl
from jax.experimental.pallas import tpu as pltpu
from jax.experimental.pallas import tpu_sc as plsc

N = 10000
E = 320000
D = 128
H = 8
C = 16
NPAD = 10048
ROWW = 136
NW = 32
B = 96
NB = 108
E_PAD = NW * NB * B
DUMMY = NPAD - 8
BLK = 1256


def _proj_body(x_ref, wl_ref, bl_ref, wr_ref, br_ref, xl_ref, xr_ref):
    xv = x_ref[...]
    xl_ref[...] = (
        jnp.dot(xv, wl_ref[...], preferred_element_type=jnp.float32) + bl_ref[...]
    ).astype(jnp.bfloat16)
    xr_ref[...] = (
        jnp.dot(xv, wr_ref[...], preferred_element_type=jnp.float32) + br_ref[...]
    ).astype(jnp.bfloat16)


def _proj(xpad, Wl, bl2, Wr, br2):
    return pl.pallas_call(
        _proj_body,
        grid=(NPAD // BLK,),
        in_specs=[
            pl.BlockSpec((BLK, D), lambda i: (i, 0)),
            pl.BlockSpec((D, D), lambda i: (0, 0)),
            pl.BlockSpec((1, D), lambda i: (0, 0)),
            pl.BlockSpec((D, D), lambda i: (0, 0)),
            pl.BlockSpec((1, D), lambda i: (0, 0)),
        ],
        out_specs=[
            pl.BlockSpec((BLK, D), lambda i: (i, 0)),
            pl.BlockSpec((BLK, D), lambda i: (i, 0)),
        ],
        out_shape=[
            jax.ShapeDtypeStruct((NPAD, D), jnp.bfloat16),
            jax.ShapeDtypeStruct((NPAD, D), jnp.bfloat16),
        ],
    )(xpad, Wl, bl2, Wr, br2)


def _edge_body(xl_h, xr_h, att_h, se_h, zeros_h, out_h,
               acc_sh, idx_i, xlb0, xlb1, xrb0, xrb1, msg, attv,
               sem_i, sem_g, sem_s):
    c = lax.axis_index("c")
    s = lax.axis_index("s")
    wid = s * 2 + c
    tr = NPAD // 16
    base_e = wid * (NB * B)
    lanes = lax.iota(jnp.int32, 16)
    zero16 = jnp.zeros((16,), jnp.float32)
    rows = ((xlb0, xrb0), (xlb1, xrb1))

    def _idx_start(b, slot):
        off = base_e + b * B
        pltpu.async_copy(
            se_h.at[:, pl.ds(off, B)], idx_i.at[slot], sem_i.at[slot])

    def _idx_wait(b, slot):
        off = base_e + b * B
        pltpu.make_async_copy(
            se_h.at[:, pl.ds(off, B)], idx_i.at[slot], sem_i.at[slot]).wait()

    HB = B // 2

    def _gather_start(slot, p):
        rxl, rxr = rows[p]
        for k in range(2):
            pltpu.async_copy(
                xl_h.at[idx_i.at[slot, 0, pl.ds(k * HB, HB)]],
                rxl.at[pl.ds(k * HB, HB)], sem_g.at[p, k])
            pltpu.async_copy(
                xr_h.at[idx_i.at[slot, 1, pl.ds(k * HB, HB)]],
                rxr.at[pl.ds(k * HB, HB)], sem_g.at[p, 2 + k])

    def _gather_wait(slot, p):
        rxl, rxr = rows[p]
        for k in range(2):
            pltpu.make_async_copy(
                xl_h.at[idx_i.at[slot, 0, pl.ds(k * HB, HB)]],
                rxl.at[pl.ds(k * HB, HB)], sem_g.at[p, k]).wait()
            pltpu.make_async_copy(
                xr_h.at[idx_i.at[slot, 1, pl.ds(k * HB, HB)]],
                rxr.at[pl.ds(k * HB, HB)], sem_g.at[p, 2 + k]).wait()

    def _scatter_start(slot):
        pltpu.async_copy(msg, acc_sh.at[idx_i.at[slot, 1]], sem_s, add=True)

    def _scatter_wait(slot):
        pltpu.make_async_copy(msg, acc_sh.at[idx_i.at[slot, 1]], sem_s).wait()

    def _compute(p):
        rxl, rxr = rows[p]

        def _head(h, hcarry):
            ae = [attv[h * C + 2 * k] for k in range(8)]
            ao = [attv[h * C + 2 * k + 1] for k in range(8)]
            wcol = jnp.full((16,), 128 + h, jnp.int32)
            hc = h * C

            def _grp(g, gcarry):
                eidx = g * 16 + lanes
                accs = [zero16, zero16, zero16, zero16]
                xs = []
                cols = []
                for k in range(8):
                    c8 = jnp.bitwise_and(lanes + k, 7)
                    coli = c8 + 8 * h
                    vl = plsc.load_gather(rxl, [eidx, coli])
                    vr = plsc.load_gather(rxr, [eidx, coli])
                    le, lo = plsc.unpack(
                        plsc.bitcast(vl, jnp.bfloat16),
                        format=plsc.PackFormat.INTERLEAVED)
                    re_, ro = plsc.unpack(
                        plsc.bitcast(vr, jnp.bfloat16),
                        format=plsc.PackFormat.INTERLEAVED)
                    ue = le + re_
                    ue = jnp.maximum(ue, 0.2 * ue)
                    uo = lo + ro
                    uo = jnp.maximum(uo, 0.2 * uo)
                    accs[k % 4] = accs[k % 4] + ue * ae[k]
                    accs[(k + 2) % 4] = accs[(k + 2) % 4] + uo * ao[k]
                    ce = 2 * c8 + hc
                    xs.append((le, lo))
                    cols.append(ce)
                w = jnp.exp((accs[0] + accs[1]) + (accs[2] + accs[3]))
                plsc.store_scatter(msg, [eidx, wcol], w)
                for k in range(8):
                    le, lo = xs[k]
                    ce = cols[k]
                    plsc.store_scatter(msg, [eidx, ce], le * w)
                    plsc.store_scatter(msg, [eidx, ce + 1], lo * w)
                return gcarry

            return lax.fori_loop(0, B // 16, _grp, hcarry)

        lax.fori_loop(0, H, _head, 0)

    _idx_start(0, 0)
    _idx_start(1, 1)
    pltpu.sync_copy(att_h, attv)
    pltpu.sync_copy(zeros_h, acc_sh.at[pl.ds(s * tr, tr)])
    _idx_wait(0, 0)
    _gather_start(0, 0)
    plsc.subcore_barrier()

    def _iter(i, carry):
        for par in range(4):
            b = i * 4 + par
            p = par & 1

            @pl.when(b + 1 < NB)
            def _():
                _idx_wait(b + 1, (par + 1) % 4)
                _gather_start((par + 1) % 4, 1 - p)

            _gather_wait(par, p)

            @pl.when(b > 0)
            def _():
                _scatter_wait((par + 3) % 4)

            _compute(p)
            _scatter_start(par)

            @pl.when(b + 2 < NB)
            def _():
                _idx_start(b + 2, (par + 2) % 4)

        return carry

    lax.fori_loop(0, NB // 4, _iter, 0)

    _scatter_wait((NB - 1) % 4)
    plsc.subcore_barrier()
    pltpu.sync_copy(
        acc_sh.at[pl.ds(s * tr, tr)],
        out_h.at[c, pl.ds(s * tr, tr)],
    )


def _edge_pass(xl, xr, attf, se, zeros):
    mesh = plsc.VectorSubcoreMesh(core_axis_name="c", subcore_axis_name="s")
    kern = pl.kernel(
        _edge_body,
        out_type=jax.ShapeDtypeStruct((2, NPAD, ROWW), jnp.float32),
        mesh=mesh,
        scratch_types=[
            pltpu.VMEM_SHARED((NPAD, ROWW), jnp.float32),
            pltpu.VMEM((4, 2, B), jnp.int32),
            pltpu.VMEM((B, D // 2), jnp.int32),
            pltpu.VMEM((B, D // 2), jnp.int32),
            pltpu.VMEM((B, D // 2), jnp.int32),
            pltpu.VMEM((B, D // 2), jnp.int32),
            pltpu.VMEM((B, ROWW), jnp.float32),
            pltpu.VMEM((D, 16), jnp.float32),
            pltpu.SemaphoreType.DMA((4,)),
            pltpu.SemaphoreType.DMA((2, 4)),
            pltpu.SemaphoreType.DMA,
        ],
        compiler_params=pltpu.CompilerParams(
            needs_layout_passes=False, use_tc_tiling_on_sc=False),
    )
    return kern(xl, xr, attf, se, zeros)


def _epi_body(agg_ref, den_ref, xi_ref, wres_ref, bres_ref, gb_ref,
              exp_ref, gam_ref, bln_ref, out_ref):
    a = agg_ref[0] + agg_ref[1]
    d8 = den_ref[0] + den_ref[1]
    dfull = jnp.dot(d8, exp_ref[...], preferred_element_type=jnp.float32)
    gat = a / (dfull + 1e-16) + gb_ref[...]
    res = (
        jnp.dot(xi_ref[...], wres_ref[...], preferred_element_type=jnp.float32)
        + bres_ref[...]
    )
    y = gat + res
    mu = jnp.mean(y, axis=-1, keepdims=True)
    yc = y - mu
    var = jnp.mean(yc * yc, axis=-1, keepdims=True)
    yn = yc * lax.rsqrt(var + 1e-5)
    yn = yn * gam_ref[...] + bln_ref[...]
    out_ref[...] = 0.5 * yn * (1.0 + lax.erf(yn * (1.0 / math.sqrt(2.0))))


def _epilogue(agg, den, xipad, wres_eff, bres_eff, gb2, expand, gam2, bln2):
    return pl.pallas_call(
        _epi_body,
        grid=(NPAD // BLK,),
        in_specs=[
            pl.BlockSpec((2, BLK, D), lambda i: (0, i, 0)),
            pl.BlockSpec((2, BLK, H), lambda i: (0, i, 0)),
            pl.BlockSpec((BLK, D), lambda i: (i, 0)),
            pl.BlockSpec((D, D), lambda i: (0, 0)),
            pl.BlockSpec((1, D), lambda i: (0, 0)),
            pl.BlockSpec((1, D), lambda i: (0, 0)),
            pl.BlockSpec((H, D), lambda i: (0, 0)),
            pl.BlockSpec((1, D), lambda i: (0, 0)),
            pl.BlockSpec((1, D), lambda i: (0, 0)),
        ],
        out_specs=pl.BlockSpec((BLK, D), lambda i: (i, 0)),
        out_shape=jax.ShapeDtypeStruct((NPAD, D), jnp.float32),
    )(agg, den, xipad, wres_eff, bres_eff, gb2, expand, gam2, bln2)


def kernel(x, x_initial, edge_index, Wl, bl, Wr, br, att, gat_bias,
           Wres, bres, beta, gamma, beta_ln):
    f32 = jnp.float32
    xpad = jnp.zeros((NPAD, D), f32).at[:N].set(x)
    xipad = jnp.zeros((NPAD, D), f32).at[:N].set(x_initial)

    loop = jnp.arange(N, dtype=jnp.int32)
    npad_e = E_PAD - (E + N)
    src = jnp.concatenate(
        [edge_index[0], loop, jnp.full((npad_e,), DUMMY, jnp.int32)])
    dst = jnp.concatenate(
        [edge_index[1], loop, jnp.full((npad_e,), DUMMY, jnp.int32)])
    se = jnp.stack([src, dst])

    xl_bf, xr_bf = _proj(xpad, Wl, bl.reshape(1, D), Wr, br.reshape(1, D))
    xl = jax.lax.bitcast_convert_type(
        xl_bf.reshape(NPAD, D // 2, 2), jnp.int32)
    xr = jax.lax.bitcast_convert_type(
        xr_bf.reshape(NPAD, D // 2, 2), jnp.int32)

    rot8 = (jnp.arange(8)[:, None] + jnp.arange(16)[None, :]) % 8
    att_e = att[:, 2 * rot8]
    att_o = att[:, 2 * rot8 + 1]
    attf = jnp.stack([att_e, att_o], axis=2).reshape(D, 16)
    zeros = jnp.zeros((NPAD // 16, ROWW), f32)
    acc = _edge_pass(xl, xr, attf, se, zeros)

    agg = acc[:, :, :D]
    den = acc[:, :, D:D + H]

    expand = jnp.repeat(jnp.eye(H, dtype=f32), C, axis=1)
    wres_eff = Wres * beta
    bres_eff = (bres * beta).reshape(1, D)

    y = _epilogue(agg, den, xipad, wres_eff, bres_eff,
                  gat_bias.reshape(1, D), expand,
                  gamma.reshape(1, D), beta_ln.reshape(1, D))
    return y[:N]


# speedup vs baseline: 1.2437x; 1.0252x over previous
import functools
import math

import jax
import jax.numpy as jnp
from jax import lax
from jax.experimental import pallas as pl
from jax.experimental.pallas import tpu as pltpu
from jax.experimental.pallas import tpu_sc as plsc

N = 10000
E = 320000
D = 128
H = 8
C = 16
NPAD = 10048
ROWW = 136
NW = 32
B = 96
NB = 108
E_PAD = NW * NB * B
DUMMY = NPAD - 8
BLK = 1256


def _proj_body(x_ref, wl_ref, bl_ref, wr_ref, br_ref, xl_ref, xr_ref):
    xv = x_ref[...]
    xl_ref[...] = (
        jnp.dot(xv, wl_ref[...], preferred_element_type=jnp.float32) + bl_ref[...]
    ).astype(jnp.bfloat16)
    xr_ref[...] = (
        jnp.dot(xv, wr_ref[...], preferred_element_type=jnp.float32) + br_ref[...]
    ).astype(jnp.bfloat16)


def _proj(xpad, Wl, bl2, Wr, br2):
    return pl.pallas_call(
        _proj_body,
        grid=(NPAD // BLK,),
        in_specs=[
            pl.BlockSpec((BLK, D), lambda i: (i, 0)),
            pl.BlockSpec((D, D), lambda i: (0, 0)),
            pl.BlockSpec((1, D), lambda i: (0, 0)),
            pl.BlockSpec((D, D), lambda i: (0, 0)),
            pl.BlockSpec((1, D), lambda i: (0, 0)),
        ],
        out_specs=[
            pl.BlockSpec((BLK, D), lambda i: (i, 0)),
            pl.BlockSpec((BLK, D), lambda i: (i, 0)),
        ],
        out_shape=[
            jax.ShapeDtypeStruct((NPAD, D), jnp.bfloat16),
            jax.ShapeDtypeStruct((NPAD, D), jnp.bfloat16),
        ],
    )(xpad, Wl, bl2, Wr, br2)


def _edge_body(xl_h, xr_h, att_h, se_h, zeros_h, out_h,
               acc_sh, idx_i, xlb0, xlb1, xrb0, xrb1, msg, attv,
               sem_i, sem_g, sem_s):
    c = lax.axis_index("c")
    s = lax.axis_index("s")
    wid = s * 2 + c
    tr = NPAD // 16
    base_e = wid * (NB * B)
    lanes = lax.iota(jnp.int32, 16)
    zero16 = jnp.zeros((16,), jnp.float32)
    rows = ((xlb0, xrb0), (xlb1, xrb1))

    def _idx_start(b, slot):
        off = base_e + b * B
        pltpu.async_copy(
            se_h.at[:, pl.ds(off, B)], idx_i.at[slot], sem_i.at[slot])

    def _idx_wait(b, slot):
        off = base_e + b * B
        pltpu.make_async_copy(
            se_h.at[:, pl.ds(off, B)], idx_i.at[slot], sem_i.at[slot]).wait()

    HB = B // 2

    def _gather_start(slot, p):
        rxl, rxr = rows[p]
        for k in range(2):
            pltpu.async_copy(
                xl_h.at[idx_i.at[slot, 0, pl.ds(k * HB, HB)]],
                rxl.at[pl.ds(k * HB, HB)], sem_g.at[p, k])
            pltpu.async_copy(
                xr_h.at[idx_i.at[slot, 1, pl.ds(k * HB, HB)]],
                rxr.at[pl.ds(k * HB, HB)], sem_g.at[p, 2 + k])

    def _gather_wait(slot, p):
        rxl, rxr = rows[p]
        for k in range(2):
            pltpu.make_async_copy(
                xl_h.at[idx_i.at[slot, 0, pl.ds(k * HB, HB)]],
                rxl.at[pl.ds(k * HB, HB)], sem_g.at[p, k]).wait()
            pltpu.make_async_copy(
                xr_h.at[idx_i.at[slot, 1, pl.ds(k * HB, HB)]],
                rxr.at[pl.ds(k * HB, HB)], sem_g.at[p, 2 + k]).wait()

    def _scatter_start(slot):
        pltpu.async_copy(msg, acc_sh.at[idx_i.at[slot, 1]], sem_s, add=True)

    def _scatter_wait(slot):
        pltpu.make_async_copy(msg, acc_sh.at[idx_i.at[slot, 1]], sem_s).wait()

    def _compute(p):
        rxl, rxr = rows[p]

        def _head(h, hcarry):
            ae = [attv[h * C + 2 * k] for k in range(8)]
            ao = [attv[h * C + 2 * k + 1] for k in range(8)]
            wcol = jnp.full((16,), 128 + h, jnp.int32)
            hc = h * C

            def _grp(g, gcarry):
                eidx = g * 16 + lanes
                accs = [zero16, zero16, zero16, zero16]
                xs = []
                cols = []
                for k in range(8):
                    c8 = jnp.bitwise_and(lanes + k, 7)
                    coli = c8 + 8 * h
                    vl = plsc.load_gather(rxl, [eidx, coli])
                    vr = plsc.load_gather(rxr, [eidx, coli])
                    le, lo = plsc.unpack(
                        plsc.bitcast(vl, jnp.bfloat16),
                        format=plsc.PackFormat.INTERLEAVED)
                    re_, ro = plsc.unpack(
                        plsc.bitcast(vr, jnp.bfloat16),
                        format=plsc.PackFormat.INTERLEAVED)
                    ue = le + re_
                    ue = jnp.maximum(ue, 0.2 * ue)
                    uo = lo + ro
                    uo = jnp.maximum(uo, 0.2 * uo)
                    accs[k % 4] = accs[k % 4] + ue * ae[k]
                    accs[(k + 2) % 4] = accs[(k + 2) % 4] + uo * ao[k]
                    ce = 2 * c8 + hc
                    xs.append((le, lo))
                    cols.append(ce)
                w = jnp.exp((accs[0] + accs[1]) + (accs[2] + accs[3]))
                plsc.store_scatter(msg, [eidx, wcol], w)
                for k in range(8):
                    le, lo = xs[k]
                    ce = cols[k]
                    plsc.store_scatter(msg, [eidx, ce], le * w)
                    plsc.store_scatter(msg, [eidx, ce + 1], lo * w)
                return gcarry

            return lax.fori_loop(0, B // 16, _grp, hcarry)

        lax.fori_loop(0, H, _head, 0)

    _idx_start(0, 0)
    _idx_start(1, 1)
    pltpu.sync_copy(att_h, attv)
    pltpu.sync_copy(zeros_h, acc_sh.at[pl.ds(s * tr, tr)])
    _idx_wait(0, 0)
    _gather_start(0, 0)
    plsc.subcore_barrier()

    def _iter(i, carry):
        for par in range(4):
            b = i * 4 + par
            p = par & 1

            @pl.when(b + 1 < NB)
            def _():
                _idx_wait(b + 1, (par + 1) % 4)
                _gather_start((par + 1) % 4, 1 - p)

            _gather_wait(par, p)

            @pl.when(b > 0)
            def _():
                _scatter_wait((par + 3) % 4)

            _compute(p)
            _scatter_start(par)

            @pl.when(b + 2 < NB)
            def _():
                _idx_start(b + 2, (par + 2) % 4)

        return carry

    lax.fori_loop(0, NB // 4, _iter, 0)

    _scatter_wait((NB - 1) % 4)
    plsc.subcore_barrier()
    pltpu.sync_copy(
        acc_sh.at[pl.ds(s * tr, tr)],
        out_h.at[c, pl.ds(s * tr, tr)],
    )


def _edge_pass(xl, xr, attf, se, zeros):
    mesh = plsc.VectorSubcoreMesh(core_axis_name="c", subcore_axis_name="s")
    kern = pl.kernel(
        _edge_body,
        out_type=jax.ShapeDtypeStruct((2, NPAD, ROWW), jnp.float32),
        mesh=mesh,
        scratch_types=[
            pltpu.VMEM_SHARED((NPAD, ROWW), jnp.float32),
            pltpu.VMEM((4, 2, B), jnp.int32),
            pltpu.VMEM((B, D // 2), jnp.int32),
            pltpu.VMEM((B, D // 2), jnp.int32),
            pltpu.VMEM((B, D // 2), jnp.int32),
            pltpu.VMEM((B, D // 2), jnp.int32),
            pltpu.VMEM((B, ROWW), jnp.float32),
            pltpu.VMEM((D, 16), jnp.float32),
            pltpu.SemaphoreType.DMA((4,)),
            pltpu.SemaphoreType.DMA((2, 4)),
            pltpu.SemaphoreType.DMA,
        ],
        compiler_params=pltpu.CompilerParams(
            needs_layout_passes=False, use_tc_tiling_on_sc=False),
    )
    return kern(xl, xr, attf, se, zeros)


def _epi_body(agg_ref, den_ref, xi_ref, wres_ref, bres_ref, gb_ref,
              exp_ref, gam_ref, bln_ref, out_ref):
    a = agg_ref[0] + agg_ref[1]
    d8 = den_ref[0] + den_ref[1]
    dfull = jnp.dot(d8, exp_ref[...], preferred_element_type=jnp.float32)
    gat = a / (dfull + 1e-16) + gb_ref[...]
    res = (
        jnp.dot(xi_ref[...], wres_ref[...], preferred_element_type=jnp.float32)
        + bres_ref[...]
    )
    y = gat + res
    mu = jnp.mean(y, axis=-1, keepdims=True)
    yc = y - mu
    var = jnp.mean(yc * yc, axis=-1, keepdims=True)
    yn = yc * lax.rsqrt(var + 1e-5)
    yn = yn * gam_ref[...] + bln_ref[...]
    out_ref[...] = 0.5 * yn * (1.0 + lax.erf(yn * (1.0 / math.sqrt(2.0))))


EBLK = 1000


def _epilogue(acc, x_initial, wres_eff, bres_eff, gb2, expand, gam2, bln2):
    return pl.pallas_call(
        _epi_body,
        grid=(N // EBLK,),
        in_specs=[
            pl.BlockSpec((2, EBLK, D), lambda i: (0, i, 0)),
            pl.BlockSpec((2, EBLK, H), lambda i: (0, i, 0)),
            pl.BlockSpec((EBLK, D), lambda i: (i, 0)),
            pl.BlockSpec((D, D), lambda i: (0, 0)),
            pl.BlockSpec((1, D), lambda i: (0, 0)),
            pl.BlockSpec((1, D), lambda i: (0, 0)),
            pl.BlockSpec((H, D), lambda i: (0, 0)),
            pl.BlockSpec((1, D), lambda i: (0, 0)),
            pl.BlockSpec((1, D), lambda i: (0, 0)),
        ],
        out_specs=pl.BlockSpec((EBLK, D), lambda i: (i, 0)),
        out_shape=jax.ShapeDtypeStruct((N, D), jnp.float32),
    )(acc, acc[:, :, D:], x_initial, wres_eff, bres_eff, gb2, expand,
      gam2, bln2)


def kernel(x, x_initial, edge_index, Wl, bl, Wr, br, att, gat_bias,
           Wres, bres, beta, gamma, beta_ln):
    f32 = jnp.float32
    xpad = jnp.zeros((NPAD, D), f32).at[:N].set(x)

    loop = jnp.arange(N, dtype=jnp.int32)
    npad_e = E_PAD - (E + N)
    src = jnp.concatenate(
        [edge_index[0], loop, jnp.full((npad_e,), DUMMY, jnp.int32)])
    dst = jnp.concatenate(
        [edge_index[1], loop, jnp.full((npad_e,), DUMMY, jnp.int32)])
    se = jnp.stack([src, dst])

    xl_bf, xr_bf = _proj(xpad, Wl, bl.reshape(1, D), Wr, br.reshape(1, D))
    xl = jax.lax.bitcast_convert_type(
        xl_bf.reshape(NPAD, D // 2, 2), jnp.int32)
    xr = jax.lax.bitcast_convert_type(
        xr_bf.reshape(NPAD, D // 2, 2), jnp.int32)

    rot8 = (jnp.arange(8)[:, None] + jnp.arange(16)[None, :]) % 8
    att_e = att[:, 2 * rot8]
    att_o = att[:, 2 * rot8 + 1]
    attf = jnp.stack([att_e, att_o], axis=2).reshape(D, 16)
    zeros = jnp.zeros((NPAD // 16, ROWW), f32)
    acc = _edge_pass(xl, xr, attf, se, zeros)

    expand = jnp.repeat(jnp.eye(H, dtype=f32), C, axis=1)
    wres_eff = Wres * beta
    bres_eff = (bres * beta).reshape(1, D)

    return _epilogue(acc, x_initial, wres_eff, bres_eff,
                     gat_bias.reshape(1, D), expand,
                     gamma.reshape(1, D), beta_ln.reshape(1, D))
